# baseline (speedup 1.0000x reference)
import jax
import jax.numpy as jnp
from jax.experimental import pallas as pl
from jax.experimental.pallas import tpu as pltpu

_BR = 1000
_NB = 8
_BAND = 125000


def _copy_body(*refs):
    ins = refs[:_NB]
    outs = refs[_NB:]
    for k in range(_NB):
        outs[k][...] = ins[k][...]


def kernel(embeddings):
    rows, dim = embeddings.shape
    grid = _BAND // _BR
    in_specs = [
        pl.BlockSpec((_BR, dim), (lambda i, k=k: (k * grid + i, 0)))
        for k in range(_NB)
    ]
    out_specs = [
        pl.BlockSpec((_BR, dim), (lambda i: (i, 0))) for _ in range(_NB)
    ]
    outs = pl.pallas_call(
        _copy_body,
        out_shape=[jax.ShapeDtypeStruct((_BAND, dim), embeddings.dtype)] * _NB,
        grid=(grid,),
        in_specs=in_specs,
        out_specs=out_specs,
    )(*([embeddings] * _NB))
    return jnp.concatenate(outs, axis=0)


# speedup vs baseline: 1.5241x; 1.5241x over previous
import jax
import jax.numpy as jnp
from jax.experimental import pallas as pl
from jax.experimental.pallas import tpu as pltpu

_BR = 5000
_N = 25
_NBUF = 2
_LOOKAHEAD = 1


def _copy_body(src, dst, buf, in_sems, out_sems):
    def in_cp(i):
        return pltpu.make_async_copy(
            src.at[:, pl.ds(i * _BR, _BR), :], buf.at[i % _NBUF],
            in_sems.at[i % _NBUF])

    def out_cp(i):
        return pltpu.make_async_copy(
            buf.at[i % _NBUF], dst.at[:, pl.ds(i * _BR, _BR), :],
            out_sems.at[i % _NBUF])

    for i in range(_LOOKAHEAD):
        in_cp(i).start()
    for i in range(_N):
        in_cp(i).wait()
        out_cp(i).start()
        nxt = i + _LOOKAHEAD
        if nxt < _N:
            if nxt >= _NBUF:
                out_cp(nxt - _NBUF).wait()
            in_cp(nxt).start()
    for i in range(max(0, _N - _NBUF), _N):
        out_cp(i).wait()


def kernel(embeddings):
    rows, dim = embeddings.shape
    v = embeddings.reshape(8, rows // 8, dim)
    out = pl.pallas_call(
        _copy_body,
        out_shape=jax.ShapeDtypeStruct(v.shape, v.dtype),
        in_specs=[pl.BlockSpec(memory_space=pl.ANY)],
        out_specs=pl.BlockSpec(memory_space=pl.ANY),
        scratch_shapes=[
            pltpu.VMEM((_NBUF, 8, _BR, dim), v.dtype),
            pltpu.SemaphoreType.DMA((_NBUF,)),
            pltpu.SemaphoreType.DMA((_NBUF,)),
        ],
    )(v)
    return out.reshape(rows, dim)
